# Initial kernel scaffold; baseline (speedup 1.0000x reference)
import jax
import jax.numpy as jnp
from jax.experimental import pallas as pl


def kernel(x_in, W_cl1, b_cl1, g1, be1, W_cl2, b_cl2, W_cl3, b_cl3, W_reg1, b_reg1, gr, br, W_cm2, b_cm2, W_cm3, b_cm3):
    raise NotImplementedError("write your pallas kernel here")


# speedup vs baseline: 4.6113x; 4.6113x over previous
import functools
import jax
import jax.numpy as jnp
from jax.experimental import pallas as pl
from jax.experimental.pallas import tpu as pltpu

CLS = 64
CH = 128


def _lrelu(x):
    return jnp.where(x >= 0, x, 0.01 * x)


def _stats_body(x_ref, wcl_ref, bcl_ref, wrg_ref, brg_ref, out_ref, acc):
    step = pl.program_id(0) * pl.num_programs(1) + pl.program_id(1)

    @pl.when(step == 0)
    def _():
        acc[...] = jnp.zeros_like(acc)

    x = x_ref[0]
    ycl = jnp.dot(wcl_ref[...], x, preferred_element_type=jnp.float32) + bcl_ref[...][:, 0:1]
    yrg = jnp.dot(wrg_ref[...], x, preferred_element_type=jnp.float32) + brg_ref[...][:, 0:1]
    s = jnp.concatenate(
        [
            jnp.sum(ycl, axis=1, keepdims=True),
            jnp.sum(ycl * ycl, axis=1, keepdims=True),
            jnp.sum(yrg, axis=1, keepdims=True),
            jnp.sum(yrg * yrg, axis=1, keepdims=True),
        ],
        axis=1,
    )
    acc[...] += s

    @pl.when(step == pl.num_programs(0) * pl.num_programs(1) - 1)
    def _():
        out_ref[...] = acc[...]


def _main_body(
    x_ref, wcl1_ref, sc1_ref, sh1_ref, wcl2_ref, bcl2_ref, wcl3_ref, bcl3_ref,
    wrg_ref, scr_ref, shr_ref, w2f_ref, b2f_ref, w3b_ref, b3_ref,
    xreal_ref, mask_ref,
):
    x = x_ref[0]
    wt = x.shape[1]

    y = jnp.dot(wcl1_ref[...], x, preferred_element_type=jnp.float32)
    h1 = _lrelu(y * sc1_ref[...][:, 0:1] + sh1_ref[...][:, 0:1])
    h2 = _lrelu(jnp.dot(wcl2_ref[...], h1, preferred_element_type=jnp.float32)
                + bcl2_ref[...][:, 0:1])
    logits = jnp.dot(wcl3_ref[...], h2, preferred_element_type=jnp.float32) + bcl3_ref[...][:, 0:1]
    cls = logits[0:CLS, :]
    m = jnp.max(cls, axis=0, keepdims=True)
    iota = jax.lax.broadcasted_iota(jnp.int32, (CLS, wt), 0)
    ind = jnp.min(jnp.where(cls == m, iota, CLS), axis=0, keepdims=True)
    mask_ref[0, 0] = _lrelu(logits[CLS : CLS + 1, :])

    yr = jnp.dot(wrg_ref[...], x, preferred_element_type=jnp.float32)
    xr = _lrelu(yr * scr_ref[...][:, 0:1] + shr_ref[...][:, 0:1])

    z = jnp.dot(w2f_ref[...], xr.astype(jnp.bfloat16),
                preferred_element_type=jnp.float32)
    z = _lrelu(z + b2f_ref[...][:, 0:1])
    y3 = jnp.dot(w3b_ref[...], z.astype(jnp.bfloat16),
                 preferred_element_type=jnp.float32) + b3_ref[...][:, 0:1]
    onehot = jax.lax.broadcasted_iota(jnp.int32, (CLS, wt), 0) == ind
    reg = jnp.sum(jnp.where(onehot, y3, 0.0), axis=0, keepdims=True)
    xreal_ref[0, 0] = (ind.astype(jnp.float32) + reg) * (1.0 / CLS)


def _col(v):
    return v.reshape(-1, 1)


@functools.partial(jax.jit, static_argnames=())
def kernel(x_in, W_cl1, b_cl1, g1, be1, W_cl2, b_cl2, W_cl3, b_cl3,
           W_reg1, b_reg1, gr, br, W_cm2, b_cm2, W_cm3, b_cm3):
    B, Cin, H, Wd = x_in.shape
    N = B * H * Wd
    x3 = x_in.reshape(B, Cin, H * Wd)
    WT = 512
    grid = (B, (H * Wd) // WT)

    stats = pl.pallas_call(
        _stats_body,
        grid=grid,
        in_specs=[
            pl.BlockSpec((1, Cin, WT), lambda b, w: (b, 0, w)),
            pl.BlockSpec((CH, Cin), lambda b, w: (0, 0)),
            pl.BlockSpec((CH, 1), lambda b, w: (0, 0)),
            pl.BlockSpec((CH, Cin), lambda b, w: (0, 0)),
            pl.BlockSpec((CH, 1), lambda b, w: (0, 0)),
        ],
        out_specs=pl.BlockSpec((CH, 4), lambda b, w: (0, 0)),
        out_shape=jax.ShapeDtypeStruct((CH, 4), jnp.float32),
        scratch_shapes=[pltpu.VMEM((CH, 4), jnp.float32)],
    )(x3, W_cl1, _col(b_cl1), W_reg1, _col(b_reg1))

    eps = 1e-5
    n = jnp.float32(N)
    mean_cl, msq_cl = stats[:, 0] / n, stats[:, 1] / n
    mean_rg, msq_rg = stats[:, 2] / n, stats[:, 3] / n
    var_cl = msq_cl - mean_cl * mean_cl
    var_rg = msq_rg - mean_rg * mean_rg
    sc1 = g1 / jnp.sqrt(var_cl + eps)
    sh1 = be1 - mean_cl * sc1 + b_cl1 * sc1
    scr = gr / jnp.sqrt(var_rg + eps)
    shr = br - mean_rg * scr + b_reg1 * scr

    Wcl3p = jnp.zeros((CH, CH), jnp.float32).at[: CLS + 1, :].set(W_cl3)
    bcl3p = jnp.zeros((CH,), jnp.float32).at[: CLS + 1].set(b_cl3)

    w2f = W_cm2.transpose(0, 2, 1).reshape(CLS * 32, CH).astype(jnp.bfloat16)
    b2f = b_cm2.reshape(CLS * 32)
    w3b = (jnp.eye(CLS, dtype=jnp.float32)[:, :, None]
           * W_cm3[None, :, :, 0]).reshape(CLS, CLS * 32).astype(jnp.bfloat16)
    b3 = b_cm3[:, 0]

    xreal, mask = pl.pallas_call(
        _main_body,
        grid=grid,
        in_specs=[
            pl.BlockSpec((1, Cin, WT), lambda b, w: (b, 0, w)),
            pl.BlockSpec((CH, Cin), lambda b, w: (0, 0)),
            pl.BlockSpec((CH, 1), lambda b, w: (0, 0)),
            pl.BlockSpec((CH, 1), lambda b, w: (0, 0)),
            pl.BlockSpec((CH, CH), lambda b, w: (0, 0)),
            pl.BlockSpec((CH, 1), lambda b, w: (0, 0)),
            pl.BlockSpec((CH, CH), lambda b, w: (0, 0)),
            pl.BlockSpec((CH, 1), lambda b, w: (0, 0)),
            pl.BlockSpec((CH, Cin), lambda b, w: (0, 0)),
            pl.BlockSpec((CH, 1), lambda b, w: (0, 0)),
            pl.BlockSpec((CH, 1), lambda b, w: (0, 0)),
            pl.BlockSpec((CLS * 32, CH), lambda b, w: (0, 0)),
            pl.BlockSpec((CLS * 32, 1), lambda b, w: (0, 0)),
            pl.BlockSpec((CLS, CLS * 32), lambda b, w: (0, 0)),
            pl.BlockSpec((CLS, 1), lambda b, w: (0, 0)),
        ],
        out_specs=[
            pl.BlockSpec((1, 1, 1, WT), lambda b, w: (b, 0, 0, w)),
            pl.BlockSpec((1, 1, 1, WT), lambda b, w: (b, 0, 0, w)),
        ],
        out_shape=[
            jax.ShapeDtypeStruct((B, 1, H, Wd), jnp.float32),
            jax.ShapeDtypeStruct((B, 1, H, Wd), jnp.float32),
        ],
    )(x3, W_cl1, _col(sc1), _col(sh1), W_cl2, _col(b_cl2), Wcl3p, _col(bcl3p),
      W_reg1, _col(scr), _col(shr), w2f, _col(b2f), w3b, _col(b3))

    return (xreal, mask)
